# Initial kernel scaffold; baseline (speedup 1.0000x reference)
import jax
import jax.numpy as jnp
from jax.experimental import pallas as pl


def kernel(x, edge_index, W1, a_src1, a_dst1, b1, W2, a_src2, a_dst2, b2, W3, a_src3, a_dst3, b3):
    raise NotImplementedError("write your pallas kernel here")


# speedup vs baseline: 7.5922x; 7.5922x over previous
import functools
import jax
import jax.numpy as jnp
from jax import lax
from jax.experimental import pallas as pl
from jax.experimental.pallas import tpu as pltpu
from jax.experimental.pallas import tpu_sc as plsc

_NODES = 10000
_E_RAW = 320000
_ROW_BLK = 1000

_NW = 32
_K = 16
_NP = 10240
_RPT = _NP // _NW
_NCH = 648
_SCH = 72
_EPT = _NCH * _K
_EP = _NW * _EPT
_DEN_R = _NP // 128


def _mm_body(x_ref, b_ref, den_ref, w_ref, asrc_ref, adst_ref,
             h_ref, s_ref, d_ref, *, act):
    x = x_ref[...]
    if act:
        x = jnp.maximum(x / (den_ref[...] + 1e-16) + b_ref[...], 0.0)
    h = jnp.dot(x, w_ref[...], preferred_element_type=jnp.float32)
    h_ref[...] = h
    s_ref[...] = jnp.dot(h, asrc_ref[...], preferred_element_type=jnp.float32)
    d_ref[...] = jnp.dot(h, adst_ref[...], preferred_element_type=jnp.float32)


def _mm(x, bias, den, W, a_src, a_dst, act):
    n, k = x.shape
    f = W.shape[1]
    out = pl.pallas_call(
        functools.partial(_mm_body, act=act),
        grid=(n // _ROW_BLK,),
        in_specs=[
            pl.BlockSpec((_ROW_BLK, k), lambda i: (i, 0)),
            pl.BlockSpec((1, k), lambda i: (0, 0)),
            pl.BlockSpec((_ROW_BLK, 1), lambda i: (i, 0)),
            pl.BlockSpec((k, f), lambda i: (0, 0)),
            pl.BlockSpec((f, 1), lambda i: (0, 0)),
            pl.BlockSpec((f, 1), lambda i: (0, 0)),
        ],
        out_specs=[
            pl.BlockSpec((_ROW_BLK, f), lambda i: (i, 0)),
            pl.BlockSpec((_ROW_BLK, 1), lambda i: (i, 0)),
            pl.BlockSpec((_ROW_BLK, 1), lambda i: (i, 0)),
        ],
        out_shape=[
            jax.ShapeDtypeStruct((n, f), jnp.float32),
            jax.ShapeDtypeStruct((n, 1), jnp.float32),
            jax.ShapeDtypeStruct((n, 1), jnp.float32),
        ],
    )(x, bias.reshape(1, k), den.reshape(n, 1), W,
      a_src.reshape(f, 1), a_dst.reshape(f, 1))
    return out[0], out[1][:, 0], out[2][:, 0]


def _lsm_body(x_ref, b_ref, den_ref, o_ref):
    x = x_ref[...] / (den_ref[...] + 1e-16) + b_ref[...]
    m = jnp.max(x, axis=1, keepdims=True)
    ex = jnp.exp(x - m)
    o_ref[...] = x - m - jnp.log(jnp.sum(ex, axis=1, keepdims=True))


def _lsm(x, bias, den):
    n, c = x.shape
    return pl.pallas_call(
        _lsm_body,
        grid=(n // _ROW_BLK,),
        in_specs=[
            pl.BlockSpec((_ROW_BLK, c), lambda i: (i, 0)),
            pl.BlockSpec((1, c), lambda i: (0, 0)),
            pl.BlockSpec((_ROW_BLK, 1), lambda i: (i, 0)),
        ],
        out_specs=pl.BlockSpec((_ROW_BLK, c), lambda i: (i, 0)),
        out_shape=jax.ShapeDtypeStruct((n, c), jnp.float32),
    )(x, bias.reshape(1, c), den.reshape(n, 1))


def _sc_w_body(s_hbm, d_hbm, src_hbm, dst_hbm, w_hbm, den_hbm,
               s_v, d_v, src_v, dst_v, w_v, den_v, idx_v, den_sh):
    wid = lax.axis_index("s") * 2 + lax.axis_index("c")
    ebase = wid * _EPT
    pltpu.sync_copy(s_hbm, s_v)
    pltpu.sync_copy(d_hbm, d_v)

    zero = jnp.zeros((16,), jnp.float32)
    for r in range(_DEN_R):
        for fc in range(8):
            den_v[r, pl.ds(fc * 16, 16)] = zero
    for r5 in range(_DEN_R // 16):
        idx_v[pl.ds(r5 * 16, 16)] = lax.iota(jnp.int32, 16) + r5 * 16

    @pl.when(wid < 10)
    def _zero_shared():
        pltpu.sync_copy(den_v.at[pl.ds(0, 8)], den_sh.at[pl.ds(wid * 8, 8)])

    def mscan(c, m):
        return jnp.maximum(m, jnp.max(s_v[pl.ds(c * 16, 16)]))
    gmax = lax.fori_loop(0, _NP // 16, mscan, jnp.float32(-3.0e38))

    def superchunk(s, carry):
        e0 = ebase + s * _SCH * _K
        pltpu.sync_copy(src_hbm.at[pl.ds(e0, _SCH * _K)], src_v)
        pltpu.sync_copy(dst_hbm.at[pl.ds(e0, _SCH * _K)], dst_v)

        def chunk(cc, carry2):
            sl = pl.ds(cc * _K, _K)
            sidx = src_v[sl]
            didx = dst_v[sl]
            s16 = plsc.load_gather(s_v, [sidx])
            d16 = plsc.load_gather(d_v, [didx])
            e = s16 + d16
            e = jnp.where(e >= 0.0, e, 0.2 * e)
            mm = gmax + d16
            mm = jnp.where(mm >= 0.0, mm, 0.2 * mm)
            w16 = jnp.exp(e - mm)
            w_v[sl] = w16
            hi = lax.shift_right_logical(didx, 7)
            lo = lax.bitwise_and(didx, 127)
            plsc.addupdate_scatter(den_v, [hi, lo], w16)
            return carry2

        lax.fori_loop(0, _SCH, chunk, 0)
        pltpu.sync_copy(w_v, w_hbm.at[pl.ds(e0, _SCH * _K)])
        return carry

    lax.fori_loop(0, _NCH // _SCH, superchunk, 0)
    plsc.subcore_barrier()
    pltpu.sync_copy(den_v, den_sh.at[idx_v], add=True)
    plsc.subcore_barrier()

    @pl.when(wid < 10)
    def _copy_out():
        pltpu.sync_copy(den_sh.at[pl.ds(wid * 8, 8)],
                        den_hbm.at[pl.ds(wid * 8, 8)])


@functools.cache
def _make_sc_w():
    return pl.kernel(
        _sc_w_body,
        mesh=plsc.VectorSubcoreMesh(core_axis_name="c", subcore_axis_name="s"),
        out_type=(jax.ShapeDtypeStruct((_EP,), jnp.float32),
                  jax.ShapeDtypeStruct((_DEN_R, 128), jnp.float32)),
        scratch_types=[
            pltpu.VMEM((_NP,), jnp.float32),
            pltpu.VMEM((_NP,), jnp.float32),
            pltpu.VMEM((_SCH * _K,), jnp.int32),
            pltpu.VMEM((_SCH * _K,), jnp.int32),
            pltpu.VMEM((_SCH * _K,), jnp.float32),
            pltpu.VMEM((_DEN_R, 128), jnp.float32),
            pltpu.VMEM((_DEN_R,), jnp.int32),
            pltpu.VMEM_SHARED((_DEN_R, 128), jnp.float32),
        ],
        compiler_params=pltpu.CompilerParams(needs_layout_passes=False),
    )


def _sc_agg_body(h_hbm, src_hbm, dst2d_hbm, alpha_hbm, out_hbm,
                 src_v, alpha_v, dst_v, rows_v, zrow_v, acc_sh, sem, *, fb):
    wid = lax.axis_index("s") * 2 + lax.axis_index("c")
    ebase = wid * _EPT

    zero = jnp.zeros((16,), jnp.float32)
    for j in range(16):
        for fc in range(fb // 16):
            zrow_v[j, pl.ds(fc * 16, 16)] = zero
    row0 = wid * _RPT
    for r in range(_RPT // 16):
        pltpu.sync_copy(zrow_v, acc_sh.at[pl.ds(row0 + r * 16, 16)])
    plsc.subcore_barrier()

    gd = lax.GatherDimensionNumbers(
        offset_dims=(), collapsed_slice_dims=(0,), start_index_map=(0,))

    def superchunk(s, carry):
        e0 = ebase + s * _SCH * _K
        pltpu.sync_copy(src_hbm.at[pl.ds(e0, _SCH * _K)], src_v)
        pltpu.sync_copy(alpha_hbm.at[pl.ds(e0, _SCH * _K)], alpha_v)
        pltpu.sync_copy(dst2d_hbm.at[pl.ds(wid * _NCH + s * _SCH, _SCH)], dst_v)

        def chunk(c, carry2):
            idx = src_v.at[pl.ds(c * _K, _K)]
            pltpu.async_copy(h_hbm.at[idx], rows_v, sem).wait()
            ach = alpha_v[pl.ds(c * _K, _K)]
            for j in range(_K):
                jidx = jnp.full((_K, 1), j, jnp.int32)
                a = lax.gather(ach, jidx, gd, (1,),
                               mode=lax.GatherScatterMode.PROMISE_IN_BOUNDS)
                for fc in range(fb // 16):
                    sl = pl.ds(fc * 16, 16)
                    rows_v[j, sl] = rows_v[j, sl] * a
            pltpu.sync_copy(rows_v, acc_sh.at[dst_v.at[c]], add=True)
            return carry2

        lax.fori_loop(0, _SCH, chunk, 0)
        return carry

    lax.fori_loop(0, _NCH // _SCH, superchunk, 0)
    plsc.subcore_barrier()
    pltpu.sync_copy(acc_sh.at[pl.ds(row0, _RPT)], out_hbm.at[pl.ds(row0, _RPT)])


@functools.cache
def _make_sc_agg(fb):
    return pl.kernel(
        functools.partial(_sc_agg_body, fb=fb),
        mesh=plsc.VectorSubcoreMesh(core_axis_name="c", subcore_axis_name="s"),
        out_type=jax.ShapeDtypeStruct((_NP, fb), jnp.float32),
        scratch_types=[
            pltpu.VMEM((_SCH * _K,), jnp.int32),
            pltpu.VMEM((_SCH * _K,), jnp.float32),
            pltpu.VMEM((_SCH, _K), jnp.int32),
            pltpu.VMEM((_K, fb), jnp.float32),
            pltpu.VMEM((16, fb), jnp.float32),
            pltpu.VMEM_SHARED((_NP, fb), jnp.float32),
            pltpu.SemaphoreType.DMA,
        ],
    )


def _sc_agg(h, src_pad, dst2d, alpha_pad):
    n, f = h.shape
    fpad = (-f) % 128
    if fpad:
        h = jnp.pad(h, ((0, 0), (0, fpad)))
    outs = []
    for f0 in range(0, f + fpad, 128):
        outs.append(_make_sc_agg(128)(h[:, f0:f0 + 128], src_pad, dst2d, alpha_pad))
    if len(outs) == 1:
        return outs[0][:_NODES, :f]
    return jnp.concatenate(outs, axis=1)[:_NODES, :f]


def kernel(x, edge_index, W1, a_src1, a_dst1, b1, W2, a_src2, a_dst2, b2, W3, a_src3, a_dst3, b3):
    src = edge_index[0].astype(jnp.int32)
    dst = edge_index[1].astype(jnp.int32)
    loop = jnp.arange(_NODES, dtype=jnp.int32)
    npad = _EP - _E_RAW - _NODES
    src_pad = jnp.concatenate([src, loop, jnp.zeros((npad,), jnp.int32)])
    dst_pad = jnp.concatenate([dst, loop, jnp.full((npad,), _NODES, jnp.int32)])
    order = jnp.argsort(dst_pad)
    order = order.reshape(_K, _EP // _K).transpose(1, 0).reshape(-1)
    src_pad = src_pad[order]
    dst_pad = dst_pad[order]
    dst2d = dst_pad.reshape(_NW * _NCH, _K)
    vpad = _NP - _NODES

    def edge_w(s, d):
        sp = jnp.concatenate([s, jnp.full((vpad,), -3.0e38, jnp.float32)])
        dp = jnp.concatenate([d, jnp.zeros((vpad,), jnp.float32)])
        w, den2d = _make_sc_w()(sp, dp, src_pad, dst_pad)
        return w, den2d.reshape(_NP)[:_NODES]

    h1, s1, d1 = _mm(x, jnp.zeros((x.shape[1],), jnp.float32),
                     jnp.ones((x.shape[0],), jnp.float32), W1, a_src1, a_dst1, False)
    w1, den1 = edge_w(s1, d1)
    o1 = _sc_agg(h1, src_pad, dst2d, w1)

    h2, s2, d2 = _mm(o1, b1, den1, W2, a_src2, a_dst2, True)
    w2, den2 = edge_w(s2, d2)
    o2 = _sc_agg(h2, src_pad, dst2d, w2)

    h3, s3, d3 = _mm(o2, b2, den2, W3, a_src3, a_dst3, True)
    w3, den3 = edge_w(s3, d3)
    o3 = _sc_agg(h3, src_pad, dst2d, w3)

    return _lsm(o3, b3, den3)


# speedup vs baseline: 12.5149x; 1.6484x over previous
import functools
import jax
import jax.numpy as jnp
from jax import lax
from jax.experimental import pallas as pl
from jax.experimental.pallas import tpu as pltpu
from jax.experimental.pallas import tpu_sc as plsc

_NODES = 10000
_E_RAW = 320000
_ROW_BLK = 1000

_NW = 32
_K = 16
_NP = 10240
_RPT = _NP // _NW
_NCH = 648
_SCH = 72
_EPT = _NCH * _K
_EP = _NW * _EPT
_DEN_R = _NP // 128


def _mm_body(x_ref, b_ref, den_ref, w_ref, asrc_ref, adst_ref,
             h_ref, s_ref, d_ref, *, act):
    x = x_ref[...]
    if act:
        x = jnp.maximum(x / (den_ref[...] + 1e-16) + b_ref[...], 0.0)
    h = jnp.dot(x, w_ref[...], preferred_element_type=jnp.float32)
    h_ref[...] = h
    s_ref[...] = jnp.dot(h, asrc_ref[...], preferred_element_type=jnp.float32)
    d_ref[...] = jnp.dot(h, adst_ref[...], preferred_element_type=jnp.float32)


def _mm(x, bias, den, W, a_src, a_dst, act):
    n, k = x.shape
    f = W.shape[1]
    out = pl.pallas_call(
        functools.partial(_mm_body, act=act),
        grid=(n // _ROW_BLK,),
        in_specs=[
            pl.BlockSpec((_ROW_BLK, k), lambda i: (i, 0)),
            pl.BlockSpec((1, k), lambda i: (0, 0)),
            pl.BlockSpec((_ROW_BLK, 1), lambda i: (i, 0)),
            pl.BlockSpec((k, f), lambda i: (0, 0)),
            pl.BlockSpec((f, 1), lambda i: (0, 0)),
            pl.BlockSpec((f, 1), lambda i: (0, 0)),
        ],
        out_specs=[
            pl.BlockSpec((_ROW_BLK, f), lambda i: (i, 0)),
            pl.BlockSpec((_ROW_BLK, 1), lambda i: (i, 0)),
            pl.BlockSpec((_ROW_BLK, 1), lambda i: (i, 0)),
        ],
        out_shape=[
            jax.ShapeDtypeStruct((n, f), jnp.float32),
            jax.ShapeDtypeStruct((n, 1), jnp.float32),
            jax.ShapeDtypeStruct((n, 1), jnp.float32),
        ],
    )(x, bias.reshape(1, k), den.reshape(n, 1), W,
      a_src.reshape(f, 1), a_dst.reshape(f, 1))
    return out[0], out[1][:, 0], out[2][:, 0]


def _lsm_body(x_ref, b_ref, den_ref, o_ref):
    x = x_ref[...] / (den_ref[...] + 1e-16) + b_ref[...]
    m = jnp.max(x, axis=1, keepdims=True)
    ex = jnp.exp(x - m)
    o_ref[...] = x - m - jnp.log(jnp.sum(ex, axis=1, keepdims=True))


def _lsm(x, bias, den):
    n, c = x.shape
    return pl.pallas_call(
        _lsm_body,
        grid=(n // _ROW_BLK,),
        in_specs=[
            pl.BlockSpec((_ROW_BLK, c), lambda i: (i, 0)),
            pl.BlockSpec((1, c), lambda i: (0, 0)),
            pl.BlockSpec((_ROW_BLK, 1), lambda i: (i, 0)),
        ],
        out_specs=pl.BlockSpec((_ROW_BLK, c), lambda i: (i, 0)),
        out_shape=jax.ShapeDtypeStruct((n, c), jnp.float32),
    )(x, bias.reshape(1, c), den.reshape(n, 1))


def _sc_w_body(s_hbm, d_hbm, src_hbm, dst_hbm, w_hbm, den_hbm,
               s_v, d_v, src_v, dst_v, w_v, den_v, idx_v, den_sh):
    wid = lax.axis_index("s") * 2 + lax.axis_index("c")
    ebase = wid * _EPT
    pltpu.sync_copy(s_hbm, s_v)
    pltpu.sync_copy(d_hbm, d_v)

    zero = jnp.zeros((16,), jnp.float32)
    for r in range(_DEN_R):
        for fc in range(8):
            den_v[r, pl.ds(fc * 16, 16)] = zero
    for r5 in range(_DEN_R // 16):
        idx_v[pl.ds(r5 * 16, 16)] = lax.iota(jnp.int32, 16) + r5 * 16

    @pl.when(wid < 10)
    def _zero_shared():
        pltpu.sync_copy(den_v.at[pl.ds(0, 8)], den_sh.at[pl.ds(wid * 8, 8)])

    def mscan(c, m):
        return jnp.maximum(m, jnp.max(s_v[pl.ds(c * 16, 16)]))
    gmax = lax.fori_loop(0, _NP // 16, mscan, jnp.float32(-3.0e38))

    def superchunk(s, carry):
        e0 = ebase + s * _SCH * _K
        pltpu.sync_copy(src_hbm.at[pl.ds(e0, _SCH * _K)], src_v)
        pltpu.sync_copy(dst_hbm.at[pl.ds(e0, _SCH * _K)], dst_v)

        def chunk(cc, carry2):
            sl = pl.ds(cc * _K, _K)
            sidx = src_v[sl]
            didx = dst_v[sl]
            s16 = plsc.load_gather(s_v, [sidx])
            d16 = plsc.load_gather(d_v, [didx])
            e = s16 + d16
            e = jnp.where(e >= 0.0, e, 0.2 * e)
            mm = gmax + d16
            mm = jnp.where(mm >= 0.0, mm, 0.2 * mm)
            w16 = jnp.exp(e - mm)
            w_v[sl] = w16
            hi = lax.shift_right_logical(didx, 7)
            lo = lax.bitwise_and(didx, 127)
            plsc.addupdate_scatter(den_v, [hi, lo], w16)
            return carry2

        lax.fori_loop(0, _SCH, chunk, 0)
        pltpu.sync_copy(w_v, w_hbm.at[pl.ds(e0, _SCH * _K)])
        return carry

    lax.fori_loop(0, _NCH // _SCH, superchunk, 0)
    plsc.subcore_barrier()
    pltpu.sync_copy(den_v, den_sh.at[idx_v], add=True)
    plsc.subcore_barrier()

    @pl.when(wid < 10)
    def _copy_out():
        pltpu.sync_copy(den_sh.at[pl.ds(wid * 8, 8)],
                        den_hbm.at[pl.ds(wid * 8, 8)])


@functools.cache
def _make_sc_w():
    return pl.kernel(
        _sc_w_body,
        mesh=plsc.VectorSubcoreMesh(core_axis_name="c", subcore_axis_name="s"),
        out_type=(jax.ShapeDtypeStruct((_EP,), jnp.float32),
                  jax.ShapeDtypeStruct((_DEN_R, 128), jnp.float32)),
        scratch_types=[
            pltpu.VMEM((_NP,), jnp.float32),
            pltpu.VMEM((_NP,), jnp.float32),
            pltpu.VMEM((_SCH * _K,), jnp.int32),
            pltpu.VMEM((_SCH * _K,), jnp.int32),
            pltpu.VMEM((_SCH * _K,), jnp.float32),
            pltpu.VMEM((_DEN_R, 128), jnp.float32),
            pltpu.VMEM((_DEN_R,), jnp.int32),
            pltpu.VMEM_SHARED((_DEN_R, 128), jnp.float32),
        ],
        compiler_params=pltpu.CompilerParams(needs_layout_passes=False),
    )


def _sc_agg_body(h_hbm, src_hbm, dst2d_hbm, alpha_hbm, out_hbm,
                 src_v, alpha_v, dst_v, rows0_v, rows1_v, zrow_v, acc_sh,
                 sem0, sem1, *, fb):
    wid = lax.axis_index("s") * 2 + lax.axis_index("c")
    ebase = wid * _EPT

    zero = jnp.zeros((16,), jnp.float32)
    for j in range(16):
        for fc in range(fb // 16):
            zrow_v[j, pl.ds(fc * 16, 16)] = zero
    row0 = wid * _RPT
    for r in range(_RPT // 16):
        pltpu.sync_copy(zrow_v, acc_sh.at[pl.ds(row0 + r * 16, 16)])
    plsc.subcore_barrier()

    gd = lax.GatherDimensionNumbers(
        offset_dims=(), collapsed_slice_dims=(0,), start_index_map=(0,))

    def process(c, buf, semx):
        pltpu.make_async_copy(h_hbm.at[src_v.at[pl.ds(c * _K, _K)]],
                              buf, semx).wait()
        ach = alpha_v[pl.ds(c * _K, _K)]
        for j in range(_K):
            jidx = jnp.full((_K, 1), j, jnp.int32)
            a = lax.gather(ach, jidx, gd, (1,),
                           mode=lax.GatherScatterMode.PROMISE_IN_BOUNDS)
            for fc in range(fb // 16):
                sl = pl.ds(fc * 16, 16)
                buf[j, sl] = buf[j, sl] * a
        pltpu.sync_copy(buf, acc_sh.at[dst_v.at[c]], add=True)

    def superchunk(s, carry):
        e0 = ebase + s * _SCH * _K
        pltpu.sync_copy(src_hbm.at[pl.ds(e0, _SCH * _K)], src_v)
        pltpu.sync_copy(alpha_hbm.at[pl.ds(e0, _SCH * _K)], alpha_v)
        pltpu.sync_copy(dst2d_hbm.at[pl.ds(wid * _NCH + s * _SCH, _SCH)], dst_v)

        pltpu.async_copy(h_hbm.at[src_v.at[pl.ds(0, _K)]], rows0_v, sem0)

        def pair(cp, carry2):
            c0 = 2 * cp
            pltpu.async_copy(h_hbm.at[src_v.at[pl.ds((c0 + 1) * _K, _K)]],
                             rows1_v, sem1)
            process(c0, rows0_v, sem0)

            @pl.when(c0 + 2 < _SCH)
            def _():
                pltpu.async_copy(h_hbm.at[src_v.at[pl.ds((c0 + 2) * _K, _K)]],
                                 rows0_v, sem0)

            process(c0 + 1, rows1_v, sem1)
            return carry2

        lax.fori_loop(0, _SCH // 2, pair, 0)
        return carry

    lax.fori_loop(0, _NCH // _SCH, superchunk, 0)
    plsc.subcore_barrier()
    pltpu.sync_copy(acc_sh.at[pl.ds(row0, _RPT)], out_hbm.at[pl.ds(row0, _RPT)])


@functools.cache
def _make_sc_agg(fb):
    return pl.kernel(
        functools.partial(_sc_agg_body, fb=fb),
        mesh=plsc.VectorSubcoreMesh(core_axis_name="c", subcore_axis_name="s"),
        out_type=jax.ShapeDtypeStruct((_NP, fb), jnp.float32),
        scratch_types=[
            pltpu.VMEM((_SCH * _K,), jnp.int32),
            pltpu.VMEM((_SCH * _K,), jnp.float32),
            pltpu.VMEM((_SCH, _K), jnp.int32),
            pltpu.VMEM((_K, fb), jnp.float32),
            pltpu.VMEM((_K, fb), jnp.float32),
            pltpu.VMEM((16, fb), jnp.float32),
            pltpu.VMEM_SHARED((_NP, fb), jnp.float32),
            pltpu.SemaphoreType.DMA,
            pltpu.SemaphoreType.DMA,
        ],
    )


def _sc_agg(h, src_pad, dst2d, alpha_pad):
    n, f = h.shape
    fpad = (-f) % 128
    if fpad:
        h = jnp.pad(h, ((0, 0), (0, fpad)))
    outs = []
    for f0 in range(0, f + fpad, 128):
        outs.append(_make_sc_agg(128)(h[:, f0:f0 + 128], src_pad, dst2d, alpha_pad))
    if len(outs) == 1:
        return outs[0][:_NODES, :f]
    return jnp.concatenate(outs, axis=1)[:_NODES, :f]


def kernel(x, edge_index, W1, a_src1, a_dst1, b1, W2, a_src2, a_dst2, b2, W3, a_src3, a_dst3, b3):
    src = edge_index[0].astype(jnp.int32)
    dst = edge_index[1].astype(jnp.int32)
    loop = jnp.arange(_NODES, dtype=jnp.int32)
    npad = _EP - _E_RAW - _NODES
    src_pad = jnp.concatenate([src, loop, jnp.zeros((npad,), jnp.int32)])
    dst_pad = jnp.concatenate([dst, loop, jnp.full((npad,), _NODES, jnp.int32)])
    order = jnp.argsort(dst_pad)
    order = order.reshape(_K, _EP // _K).transpose(1, 0).reshape(-1)
    src_pad = src_pad[order]
    dst_pad = dst_pad[order]
    dst2d = dst_pad.reshape(_NW * _NCH, _K)
    vpad = _NP - _NODES

    def edge_w(s, d):
        sp = jnp.concatenate([s, jnp.full((vpad,), -3.0e38, jnp.float32)])
        dp = jnp.concatenate([d, jnp.zeros((vpad,), jnp.float32)])
        w, den2d = _make_sc_w()(sp, dp, src_pad, dst_pad)
        return w, den2d.reshape(_NP)[:_NODES]

    h1, s1, d1 = _mm(x, jnp.zeros((x.shape[1],), jnp.float32),
                     jnp.ones((x.shape[0],), jnp.float32), W1, a_src1, a_dst1, False)
    w1, den1 = edge_w(s1, d1)
    o1 = _sc_agg(h1, src_pad, dst2d, w1)

    h2, s2, d2 = _mm(o1, b1, den1, W2, a_src2, a_dst2, True)
    w2, den2 = edge_w(s2, d2)
    o2 = _sc_agg(h2, src_pad, dst2d, w2)

    h3, s3, d3 = _mm(o2, b2, den2, W3, a_src3, a_dst3, True)
    w3, den3 = edge_w(s3, d3)
    o3 = _sc_agg(h3, src_pad, dst2d, w3)

    return _lsm(o3, b3, den3)
